# baseline (speedup 1.0000x reference)
import functools

import jax
import jax.numpy as jnp
from jax import lax
from jax.experimental import pallas as pl
from jax.experimental.pallas import tpu as pltpu
from jax.experimental.pallas import tpu_sc as plsc

N = 10000
NP = 10240
E = 160000
EP = 163840
H = 512
G = 64
NCLS = 4
BM = 256
NBLK = NP // BM
RPT = NP // 16

_mesh = plsc.VectorSubcoreMesh(core_axis_name="c", subcore_axis_name="s",
                               num_cores=2, num_subcores=16)


@functools.partial(
    pl.kernel,
    out_type=jax.ShapeDtypeStruct((2 * NP, 128), jnp.float32),
    mesh=_mesh,
    scratch_types=[
        pltpu.VMEM((40, 128), jnp.int32),
        pltpu.VMEM((128, 128), jnp.float32),
        pltpu.VMEM_SHARED((NP, 128), jnp.float32),
    ],
)
def _edge_const(ea_hbm, dst_hbm, z_hbm, out_hbm, dstv, eav, acc):
    cid = lax.axis_index("c")
    sid = lax.axis_index("s")
    w = cid * 16 + sid
    r0 = sid * RPT
    pltpu.sync_copy(z_hbm, acc.at[pl.ds(r0, RPT)])
    pltpu.sync_copy(dst_hbm.at[w], dstv)
    plsc.subcore_barrier()

    @pl.loop(0, 40)
    def _(j):
        pltpu.sync_copy(ea_hbm.at[w * 40 + j], eav)
        pltpu.sync_copy(eav, acc.at[dstv.at[j]], add=True)

    plsc.subcore_barrier()
    pltpu.sync_copy(acc.at[pl.ds(r0, RPT)],
                    out_hbm.at[pl.ds(cid * NP + r0, RPT)])


def _make_spmm(c_in):
    half = c_in // 2
    nbuf = 3
    nblk = 128

    @functools.partial(
        pl.kernel,
        out_type=jax.ShapeDtypeStruct((c_in, NP, 128), jnp.float32),
        mesh=_mesh,
        scratch_types=(
            [pltpu.VMEM((80,), jnp.int32) for _ in range(nbuf)]
            + [pltpu.VMEM((nblk, 80), jnp.int32)]
            + [pltpu.VMEM((80, 128), jnp.float32) for _ in range(nbuf)]
            + [pltpu.VMEM_SHARED((NP, 128), jnp.float32)]
            + [pltpu.SemaphoreType.DMA for _ in range(3 * nbuf)]
        ),
    )
    def spmm(*args):
        tables = args[:c_in]
        src_hbm, dst_hbm, z_hbm, out_hbm = args[c_in:c_in + 4]
        rest = args[c_in + 4:]
        sidx = rest[:nbuf]
        dstv = rest[nbuf]
        gbufs = rest[nbuf + 1:2 * nbuf + 1]
        acc = rest[2 * nbuf + 1]
        gsems = rest[2 * nbuf + 2:3 * nbuf + 2]
        isems = rest[3 * nbuf + 2:4 * nbuf + 2]
        ssems = rest[4 * nbuf + 2:5 * nbuf + 2]
        cid = lax.axis_index("c")
        sid = lax.axis_index("s")
        r0 = sid * RPT
        pltpu.sync_copy(dst_hbm.at[sid], dstv)

        def idx_load(j, b):
            return pltpu.make_async_copy(src_hbm.at[sid * nblk + j],
                                         sidx[b], isems[b])

        def do_chunk(c):
            pltpu.sync_copy(z_hbm, acc.at[pl.ds(r0, RPT)])
            plsc.subcore_barrier()

            def gather(j_unused, b):
                return pltpu.make_async_copy(
                    tables[c].at[sidx[b]], gbufs[b], gsems[b])

            def scat_start(j, b):
                pltpu.async_copy(gbufs[b], acc.at[dstv.at[j]], ssems[b],
                                 add=True)

            def scat_wait(j, b):
                pltpu.make_async_copy(gbufs[b], acc.at[dstv.at[j]],
                                      ssems[b]).wait()

            for b in range(nbuf):
                idx_load(b, b).start()
            idx_load(0, 0).wait()
            gather(0, 0).start()

            main = (nblk // nbuf) * nbuf

            def step(j, u):
                nb = (u + 1) % nbuf
                gather(j, u).wait()
                scat_start(j, u)

                @pl.when(j < nblk - nbuf)
                def _():
                    idx_load(j + nbuf, u).start()

                @pl.when(j < nblk - 1)
                def _():
                    idx_load(j + 1, nb).wait()

                    @pl.when(j >= nbuf - 1)
                    def _():
                        scat_wait(j + 1 - nbuf, nb)

                    gather(j + 1, nb).start()

            @pl.loop(0, main // nbuf)
            def _(t):
                for u in range(nbuf):
                    step(nbuf * t + u, u)

            for j in range(main, nblk):
                step(j, j % nbuf)

            for u in range(nbuf):
                jj = nblk - nbuf + u
                scat_wait(jj, jj % nbuf)
            plsc.subcore_barrier()
            pltpu.sync_copy(acc.at[pl.ds(r0, RPT)],
                            out_hbm.at[c, pl.ds(r0, RPT)])

        @pl.when(cid == 0)
        def _():
            for cc in range(half):
                do_chunk(cc)

        @pl.when(cid == 1)
        def _():
            for cc in range(half):
                do_chunk(half + cc)

    return spmm


def _make_layer(c_in):
    def body(h_ref, s_ref, ea_ref, wi_ref, wj_ref, we_ref, o_ref):
        ea = ea_ref[0] + ea_ref[1]
        deg = ea[:, 16:17]
        acc = jnp.dot(ea, we_ref[...], preferred_element_type=jnp.float32)
        for c in range(c_in):
            acc = acc + jnp.dot(deg * h_ref[c], wi_ref[c],
                                preferred_element_type=jnp.float32)
            acc = acc + jnp.dot(s_ref[c], wj_ref[c],
                                preferred_element_type=jnp.float32)
        hout = jnp.maximum(acc, 0.0)
        for cp in range(4):
            o_ref[cp] = hout[:, cp * 128:(cp + 1) * 128]

    return pl.pallas_call(
        body,
        grid=(NBLK,),
        in_specs=[
            pl.BlockSpec((c_in, BM, 128), lambda m: (0, m, 0)),
            pl.BlockSpec((c_in, BM, 128), lambda m: (0, m, 0)),
            pl.BlockSpec((2, BM, 128), lambda m: (0, m, 0)),
            pl.BlockSpec((c_in, 128, H), lambda m: (0, 0, 0)),
            pl.BlockSpec((c_in, 128, H), lambda m: (0, 0, 0)),
            pl.BlockSpec((128, H), lambda m: (0, 0)),
        ],
        out_specs=pl.BlockSpec((4, BM, 128), lambda m: (0, m, 0)),
        out_shape=jax.ShapeDtypeStruct((4, NP, 128), jnp.float32),
    )


def _pool_body(h_ref, b_ref, wl_ref, bl_ref, o_ref, accs, accc):
    m = pl.program_id(0)

    @pl.when(m == 0)
    def _():
        accs[...] = jnp.zeros_like(accs)
        accc[...] = jnp.zeros_like(accc)

    bvals = b_ref[0, 0, :]
    oh = (bvals[:, None] == lax.broadcasted_iota(jnp.int32, (BM, G), 1)
          ).astype(jnp.float32)
    hblk = jnp.concatenate([h_ref[c] for c in range(4)], axis=1)
    accs[...] += lax.dot_general(oh, hblk, (((0,), (0,)), ((), ())),
                                 preferred_element_type=jnp.float32)
    accc[...] += lax.dot_general(oh, jnp.ones((BM, 128), jnp.float32),
                                 (((0,), (0,)), ((), ())),
                                 preferred_element_type=jnp.float32)

    @pl.when(m == NBLK - 1)
    def _():
        counts = jnp.maximum(accc[:, 0:1], 1.0)
        pooled = accs[...] / counts
        logits = jnp.dot(pooled, wl_ref[...],
                         preferred_element_type=jnp.float32) + bl_ref[0:1, :]
        colid = lax.broadcasted_iota(jnp.int32, (G, 128), 1)
        lm = jnp.where(colid < NCLS, logits, -1e30)
        mx = jnp.max(lm, axis=1, keepdims=True)
        se = jnp.sum(jnp.exp(lm - mx), axis=1, keepdims=True)
        o_ref[...] = lm - mx - jnp.log(se)


_pool = pl.pallas_call(
    _pool_body,
    grid=(NBLK,),
    in_specs=[
        pl.BlockSpec((4, BM, 128), lambda m: (0, m, 0)),
        pl.BlockSpec((1, 1, BM), lambda m: (m, 0, 0)),
        pl.BlockSpec((H, 128), lambda m: (0, 0)),
        pl.BlockSpec((1, 128), lambda m: (0, 0)),
    ],
    out_specs=pl.BlockSpec((G, 128), lambda m: (0, 0)),
    out_shape=jax.ShapeDtypeStruct((G, 128), jnp.float32),
    scratch_shapes=[
        pltpu.VMEM((G, H), jnp.float32),
        pltpu.VMEM((G, 128), jnp.float32),
    ],
)


def _prep_w(W, b, d):
    ci = d // 128
    wi = jnp.stack([W[:, c * 128:(c + 1) * 128].T for c in range(ci)])
    wj = jnp.stack([W[:, d + c * 128:d + (c + 1) * 128].T for c in range(ci)])
    we = jnp.zeros((128, H), jnp.float32).at[:16].set(W[:, 2 * d:].T).at[16].set(b)
    return wi, wj, we


def kernel(x, edge_index, edge_attr, batch, W0, b0, W1, b1, W2, b2, Wl, bl):
    f32 = jnp.float32
    src = edge_index[0]
    dst = edge_index[1]
    pad_e = EP - E
    src_p = jnp.concatenate([src, jnp.zeros((pad_e,), jnp.int32)])
    dst_p = jnp.concatenate([dst, jnp.full((pad_e,), NP - 1, jnp.int32)])
    src2 = src_p.reshape(2048, 80)
    dst3 = dst_p.reshape(16, 128, 80)
    dst3a = dst_p.reshape(32, 40, 128)
    ea4 = (jnp.zeros((EP, 128), f32)
           .at[:E, :16].set(edge_attr)
           .at[:, 16].set(1.0)
           .reshape(1280, 128, 128))
    z128 = jnp.zeros((RPT, 128), f32)

    eagg2 = _edge_const(ea4, dst3a, z128).reshape(2, NP, 128)

    x_pad = jnp.zeros((NP, 256), f32).at[:N].set(x)
    h = jnp.stack([x_pad[:, :128], x_pad[:, 128:]])

    for W, b in ((W0, b0), (W1, b1), (W2, b2)):
        ci = h.shape[0]
        tables = [h[c] for c in range(ci)]
        s = _make_spmm(ci)(*tables, src2, dst3, z128)
        wi, wj, we = _prep_w(W, b, ci * 128)
        h = _make_layer(ci)(h, s, eagg2, wi, wj, we)

    batch3 = (jnp.full((NP,), G, jnp.int32).at[:N].set(batch)
              .reshape(NBLK, 1, BM))
    wl_pad = jnp.zeros((H, 128), f32).at[:, :NCLS].set(Wl.T)
    bl_pad = jnp.zeros((1, 128), f32).at[0, :NCLS].set(bl)
    out128 = _pool(h, batch3, wl_pad, bl_pad)
    return out128[:, :NCLS]


# speedup vs baseline: 1.1053x; 1.1053x over previous
import functools

import jax
import jax.numpy as jnp
from jax import lax
from jax.experimental import pallas as pl
from jax.experimental.pallas import tpu as pltpu
from jax.experimental.pallas import tpu_sc as plsc

N = 10000
NP = 10240
E = 160000
EP = 163840
H = 512
G = 64
NCLS = 4
BM = 256
NBLK = NP // BM
RPT = NP // 16

_mesh = plsc.VectorSubcoreMesh(core_axis_name="c", subcore_axis_name="s",
                               num_cores=2, num_subcores=16)


@functools.partial(
    pl.kernel,
    out_type=jax.ShapeDtypeStruct((2 * NP, 128), jnp.float32),
    mesh=_mesh,
    scratch_types=[
        pltpu.VMEM((40, 128), jnp.int32),
        pltpu.VMEM((128, 128), jnp.float32),
        pltpu.VMEM_SHARED((NP, 128), jnp.float32),
    ],
)
def _edge_const(ea_hbm, dst_hbm, z_hbm, out_hbm, dstv, eav, acc):
    cid = lax.axis_index("c")
    sid = lax.axis_index("s")
    w = cid * 16 + sid
    r0 = sid * RPT
    pltpu.sync_copy(z_hbm, acc.at[pl.ds(r0, RPT)])
    pltpu.sync_copy(dst_hbm.at[w], dstv)
    plsc.subcore_barrier()

    @pl.loop(0, 40)
    def _(j):
        pltpu.sync_copy(ea_hbm.at[w * 40 + j], eav)
        pltpu.sync_copy(eav, acc.at[dstv.at[j]], add=True)

    plsc.subcore_barrier()
    pltpu.sync_copy(acc.at[pl.ds(r0, RPT)],
                    out_hbm.at[pl.ds(cid * NP + r0, RPT)])


def _make_spmm(c_in):
    half = c_in // 2
    nbuf = 2
    nblk = 80

    @functools.partial(
        pl.kernel,
        out_type=jax.ShapeDtypeStruct((c_in, NP, 128), jnp.float32),
        mesh=_mesh,
        scratch_types=(
            [pltpu.VMEM((128,), jnp.int32) for _ in range(nbuf)]
            + [pltpu.VMEM((nblk, 128), jnp.int32)]
            + [pltpu.VMEM((128, 128), jnp.float32) for _ in range(nbuf)]
            + [pltpu.VMEM_SHARED((NP, 128), jnp.float32)]
            + [pltpu.SemaphoreType.DMA for _ in range(3 * nbuf)]
        ),
    )
    def spmm(*args):
        tables = args[:c_in]
        src_hbm, dst_hbm, z_hbm, out_hbm = args[c_in:c_in + 4]
        rest = args[c_in + 4:]
        sidx = rest[:nbuf]
        dstv = rest[nbuf]
        gbufs = rest[nbuf + 1:2 * nbuf + 1]
        acc = rest[2 * nbuf + 1]
        gsems = rest[2 * nbuf + 2:3 * nbuf + 2]
        isems = rest[3 * nbuf + 2:4 * nbuf + 2]
        ssems = rest[4 * nbuf + 2:5 * nbuf + 2]
        cid = lax.axis_index("c")
        sid = lax.axis_index("s")
        r0 = sid * RPT
        pltpu.sync_copy(dst_hbm.at[sid], dstv)

        def idx_load(j, b):
            return pltpu.make_async_copy(src_hbm.at[sid * nblk + j],
                                         sidx[b], isems[b])

        def do_chunk(c):
            pltpu.sync_copy(z_hbm, acc.at[pl.ds(r0, RPT)])
            plsc.subcore_barrier()

            def gather(j_unused, b):
                return pltpu.make_async_copy(
                    tables[c].at[sidx[b]], gbufs[b], gsems[b])

            def scat_start(j, b):
                pltpu.async_copy(gbufs[b], acc.at[dstv.at[j]], ssems[b],
                                 add=True)

            def scat_wait(j, b):
                pltpu.make_async_copy(gbufs[b], acc.at[dstv.at[j]],
                                      ssems[b]).wait()

            for b in range(nbuf):
                idx_load(b, b).start()
            idx_load(0, 0).wait()
            gather(0, 0).start()

            main = (nblk // nbuf) * nbuf

            def step(j, u):
                nb = (u + 1) % nbuf
                gather(j, u).wait()
                scat_start(j, u)

                @pl.when(j < nblk - nbuf)
                def _():
                    idx_load(j + nbuf, u).start()

                @pl.when(j < nblk - 1)
                def _():
                    idx_load(j + 1, nb).wait()

                    @pl.when(j >= nbuf - 1)
                    def _():
                        scat_wait(j + 1 - nbuf, nb)

                    gather(j + 1, nb).start()

            @pl.loop(0, main // nbuf)
            def _(t):
                for u in range(nbuf):
                    step(nbuf * t + u, u)

            for j in range(main, nblk):
                step(j, j % nbuf)

            for u in range(nbuf):
                jj = nblk - nbuf + u
                scat_wait(jj, jj % nbuf)
            plsc.subcore_barrier()
            pltpu.sync_copy(acc.at[pl.ds(r0, RPT)],
                            out_hbm.at[c, pl.ds(r0, RPT)])

        @pl.when(cid == 0)
        def _():
            for cc in range(half):
                do_chunk(cc)

        @pl.when(cid == 1)
        def _():
            for cc in range(half):
                do_chunk(half + cc)

    return spmm


def _make_layer(c_in):
    def body(h_ref, s_ref, ea_ref, wi_ref, wj_ref, we_ref, o_ref):
        ea = ea_ref[0] + ea_ref[1]
        deg = ea[:, 16:17]
        acc = jnp.dot(ea, we_ref[...], preferred_element_type=jnp.float32)
        for c in range(c_in):
            acc = acc + jnp.dot(deg * h_ref[c], wi_ref[c],
                                preferred_element_type=jnp.float32)
            acc = acc + jnp.dot(s_ref[c], wj_ref[c],
                                preferred_element_type=jnp.float32)
        hout = jnp.maximum(acc, 0.0)
        for cp in range(4):
            o_ref[cp] = hout[:, cp * 128:(cp + 1) * 128]

    return pl.pallas_call(
        body,
        grid=(NBLK,),
        in_specs=[
            pl.BlockSpec((c_in, BM, 128), lambda m: (0, m, 0)),
            pl.BlockSpec((c_in, BM, 128), lambda m: (0, m, 0)),
            pl.BlockSpec((2, BM, 128), lambda m: (0, m, 0)),
            pl.BlockSpec((c_in, 128, H), lambda m: (0, 0, 0)),
            pl.BlockSpec((c_in, 128, H), lambda m: (0, 0, 0)),
            pl.BlockSpec((128, H), lambda m: (0, 0)),
        ],
        out_specs=pl.BlockSpec((4, BM, 128), lambda m: (0, m, 0)),
        out_shape=jax.ShapeDtypeStruct((4, NP, 128), jnp.float32),
    )


def _pool_body(h_ref, b_ref, wl_ref, bl_ref, o_ref, accs, accc):
    m = pl.program_id(0)

    @pl.when(m == 0)
    def _():
        accs[...] = jnp.zeros_like(accs)
        accc[...] = jnp.zeros_like(accc)

    bvals = b_ref[0, 0, :]
    oh = (bvals[:, None] == lax.broadcasted_iota(jnp.int32, (BM, G), 1)
          ).astype(jnp.float32)
    hblk = jnp.concatenate([h_ref[c] for c in range(4)], axis=1)
    accs[...] += lax.dot_general(oh, hblk, (((0,), (0,)), ((), ())),
                                 preferred_element_type=jnp.float32)
    accc[...] += lax.dot_general(oh, jnp.ones((BM, 128), jnp.float32),
                                 (((0,), (0,)), ((), ())),
                                 preferred_element_type=jnp.float32)

    @pl.when(m == NBLK - 1)
    def _():
        counts = jnp.maximum(accc[:, 0:1], 1.0)
        pooled = accs[...] / counts
        logits = jnp.dot(pooled, wl_ref[...],
                         preferred_element_type=jnp.float32) + bl_ref[0:1, :]
        colid = lax.broadcasted_iota(jnp.int32, (G, 128), 1)
        lm = jnp.where(colid < NCLS, logits, -1e30)
        mx = jnp.max(lm, axis=1, keepdims=True)
        se = jnp.sum(jnp.exp(lm - mx), axis=1, keepdims=True)
        o_ref[...] = lm - mx - jnp.log(se)


_pool = pl.pallas_call(
    _pool_body,
    grid=(NBLK,),
    in_specs=[
        pl.BlockSpec((4, BM, 128), lambda m: (0, m, 0)),
        pl.BlockSpec((1, 1, BM), lambda m: (m, 0, 0)),
        pl.BlockSpec((H, 128), lambda m: (0, 0)),
        pl.BlockSpec((1, 128), lambda m: (0, 0)),
    ],
    out_specs=pl.BlockSpec((G, 128), lambda m: (0, 0)),
    out_shape=jax.ShapeDtypeStruct((G, 128), jnp.float32),
    scratch_shapes=[
        pltpu.VMEM((G, H), jnp.float32),
        pltpu.VMEM((G, 128), jnp.float32),
    ],
)


def _prep_w(W, b, d):
    ci = d // 128
    wi = jnp.stack([W[:, c * 128:(c + 1) * 128].T for c in range(ci)])
    wj = jnp.stack([W[:, d + c * 128:d + (c + 1) * 128].T for c in range(ci)])
    we = jnp.zeros((128, H), jnp.float32).at[:16].set(W[:, 2 * d:].T).at[16].set(b)
    return wi, wj, we


def kernel(x, edge_index, edge_attr, batch, W0, b0, W1, b1, W2, b2, Wl, bl):
    f32 = jnp.float32
    src = edge_index[0]
    dst = edge_index[1]
    pad_e = EP - E
    src_p = jnp.concatenate([src, jnp.zeros((pad_e,), jnp.int32)])
    dst_p = jnp.concatenate([dst, jnp.full((pad_e,), NP - 1, jnp.int32)])
    src2 = src_p.reshape(1280, 128)
    dst3 = dst_p.reshape(16, 80, 128)
    dst3a = dst_p.reshape(32, 40, 128)
    ea4 = (jnp.zeros((EP, 128), f32)
           .at[:E, :16].set(edge_attr)
           .at[:, 16].set(1.0)
           .reshape(1280, 128, 128))
    z128 = jnp.zeros((RPT, 128), f32)

    eagg2 = _edge_const(ea4, dst3a, z128).reshape(2, NP, 128)

    x_pad = jnp.zeros((NP, 256), f32).at[:N].set(x)
    h = jnp.stack([x_pad[:, :128], x_pad[:, 128:]])

    for W, b in ((W0, b0), (W1, b1), (W2, b2)):
        ci = h.shape[0]
        tables = [h[c] for c in range(ci)]
        s = _make_spmm(ci)(*tables, src2, dst3, z128)
        wi, wj, we = _prep_w(W, b, ci * 128)
        h = _make_layer(ci)(h, s, eagg2, wi, wj, we)

    batch3 = (jnp.full((NP,), G, jnp.int32).at[:N].set(batch)
              .reshape(NBLK, 1, BM))
    wl_pad = jnp.zeros((H, 128), f32).at[:, :NCLS].set(Wl.T)
    bl_pad = jnp.zeros((1, 128), f32).at[0, :NCLS].set(bl)
    out128 = _pool(h, batch3, wl_pad, bl_pad)
    return out128[:, :NCLS]
